# baseline (speedup 1.0000x reference)
import jax
import jax.numpy as jnp
from jax.experimental import pallas as pl
from jax.experimental.pallas import tpu as pltpu

L = 50
OUT = 10
TB = 16384
CHUNKS = 4


def _gate_kernel(x_ref, w_ref, b_ref, o_ref):
    tb = x_ref.shape[0] * 8
    xr = x_ref[...].reshape(tb, L)
    yT = jax.lax.dot_general(
        w_ref[...], xr, (((0,), (1,)), ((), ())),
        preferred_element_type=jnp.float32)
    yT = jax.nn.sigmoid(yT + b_ref[...])
    eT = jnp.exp(yT)
    denomT = jax.lax.dot_general(
        jnp.ones((OUT, OUT), jnp.float32), eT, (((1,), (0,)), ((), ())),
        preferred_element_type=jnp.float32)
    rT = eT * pl.reciprocal(denomT, approx=True)
    r = jax.lax.dot_general(
        rT, jnp.eye(OUT, dtype=jnp.float32), (((0,), (0,)), ((), ())),
        preferred_element_type=jnp.float32)
    o_ref[...] = r.reshape(tb // 8, 8, OUT)


def _run_chunk(xc, w_fused, b_t):
    bc = xc.shape[0]
    tb = bc if bc <= TB else TB
    grid = (pl.cdiv(bc, tb),)
    x3 = xc.reshape(bc // 8, 8, L)
    out = pl.pallas_call(
        _gate_kernel,
        out_shape=jax.ShapeDtypeStruct((bc // 8, 8, OUT), jnp.float32),
        grid=grid,
        in_specs=[
            pl.BlockSpec((tb // 8, 8, L), lambda i: (i, 0, 0)),
            pl.BlockSpec((L, OUT), lambda i: (0, 0)),
            pl.BlockSpec((OUT, 1), lambda i: (0, 0)),
        ],
        out_specs=pl.BlockSpec((tb // 8, 8, OUT), lambda i: (i, 0, 0)),
        compiler_params=pltpu.CompilerParams(
            dimension_semantics=("parallel",)),
    )(x3, w_fused, b_t)
    return out.reshape(bc, 1, OUT)


def kernel(x, w_fused, b_fused):
    B = x.shape[0]
    assert x.shape[1] == 1 and x.shape[2] == L
    x = x.astype(jnp.float32)
    w_fused = w_fused.astype(jnp.float32)
    b_t = b_fused.astype(jnp.float32).reshape(OUT, 1)

    n_chunks = CHUNKS if (B % (CHUNKS * 8) == 0 and B >= CHUNKS * TB) else 1
    bc = B // n_chunks
    if n_chunks == 1:
        return _run_chunk(x, w_fused, b_t)
    outs = [_run_chunk(jax.lax.slice_in_dim(x, i * bc, (i + 1) * bc, axis=0),
                       w_fused, b_t)
            for i in range(n_chunks)]
    return jnp.concatenate(outs, axis=0)


# speedup vs baseline: 2.3946x; 2.3946x over previous
import jax
import jax.numpy as jnp
from jax.experimental import pallas as pl
from jax.experimental.pallas import tpu as pltpu

L = 50
OUT = 10
TB = 16384


def _gate_kernel(x_ref, w_ref, b_ref, o_ref):
    tb = x_ref.shape[0] * 8
    xr = x_ref[...].reshape(tb, L)
    yT = jax.lax.dot_general(
        w_ref[...], xr, (((0,), (1,)), ((), ())),
        preferred_element_type=jnp.float32)
    yT = jax.nn.sigmoid(yT + b_ref[...])
    eT = jnp.exp(yT)
    denomT = jax.lax.dot_general(
        jnp.ones((OUT, OUT), jnp.float32), eT, (((1,), (0,)), ((), ())),
        preferred_element_type=jnp.float32)
    rT = eT * pl.reciprocal(denomT, approx=True)
    r = jax.lax.dot_general(
        rT, jnp.eye(OUT, dtype=jnp.float32), (((0,), (0,)), ((), ())),
        preferred_element_type=jnp.float32)
    o_ref[...] = r.reshape(tb // 8, 8, OUT)


def kernel(x, w_fused, b_fused):
    B = x.shape[0]
    assert x.shape[1] == 1 and x.shape[2] == L
    x = x.astype(jnp.float32)
    w_fused = w_fused.astype(jnp.float32)
    b_fused = b_fused.astype(jnp.float32)

    tb = B if B <= TB else TB
    grid = (pl.cdiv(B, tb),)

    x3 = x.reshape(B // 8, 8, L)

    out = pl.pallas_call(
        _gate_kernel,
        out_shape=jax.ShapeDtypeStruct((B // 8, 8, OUT), jnp.float32),
        grid=grid,
        in_specs=[
            pl.BlockSpec((tb // 8, 8, L), lambda i: (i, 0, 0)),
            pl.BlockSpec((L, OUT), lambda i: (0, 0)),
            pl.BlockSpec((OUT, 1), lambda i: (0, 0)),
        ],
        out_specs=pl.BlockSpec((tb // 8, 8, OUT), lambda i: (i, 0, 0)),
        compiler_params=pltpu.CompilerParams(
            dimension_semantics=("parallel",)),
    )(x3, w_fused, b_fused.reshape(OUT, 1))

    return out.reshape(B, 1, OUT)
